# Initial kernel scaffold; baseline (speedup 1.0000x reference)
import jax
import jax.numpy as jnp
from jax.experimental import pallas as pl


def kernel(input_ids, embed_tokens):
    raise NotImplementedError("write your pallas kernel here")


# speedup vs baseline: 1.6227x; 1.6227x over previous
import functools

import jax
import jax.numpy as jnp
from jax import lax
from jax.experimental import pallas as pl
from jax.experimental.pallas import tpu as pltpu
from jax.experimental.pallas import tpu_sc as plsc

_NC = 2
_NS = 16
_NW = _NC * _NS

_CHUNK = 32


@functools.partial(jax.jit, static_argnums=())
def _gather_rows(table, ids):
    B = ids.shape[0]
    V, D = table.shape
    b_per_w = B // _NW
    num_ch = b_per_w // _CHUNK

    mesh = plsc.VectorSubcoreMesh(core_axis_name="c", subcore_axis_name="s")

    @functools.partial(
        pl.kernel,
        mesh=mesh,
        out_type=jax.ShapeDtypeStruct((B, D), jnp.float32),
        scratch_types=[
            pltpu.VMEM((b_per_w,), jnp.int32),
            pltpu.VMEM((_CHUNK, D), jnp.float32),
            pltpu.SemaphoreType.DMA,
        ],
    )
    def k(table_hbm, idx_hbm, out_hbm, idx_v, rows_v, sem):
        wid = lax.axis_index("s") * _NC + lax.axis_index("c")
        base = wid * b_per_w
        pltpu.sync_copy(idx_hbm.at[pl.ds(base, b_per_w)], idx_v)

        def body(i, carry):
            off = i * _CHUNK
            pltpu.async_copy(
                table_hbm.at[idx_v.at[pl.ds(off, _CHUNK)]], rows_v, sem
            ).wait()
            pltpu.sync_copy(rows_v, out_hbm.at[pl.ds(base + off, _CHUNK)])
            return carry

        lax.fori_loop(0, num_ch, body, 0)

    return k(table, ids)


def kernel(input_ids, embed_tokens):
    in_shape = input_ids.shape
    ids = input_ids.reshape(-1)
    out = _gather_rows(embed_tokens, ids)
    return out.reshape(in_shape + (embed_tokens.shape[1],))


# speedup vs baseline: 1.7682x; 1.0897x over previous
import functools

import jax
import jax.numpy as jnp
from jax import lax
from jax.experimental import pallas as pl
from jax.experimental.pallas import tpu as pltpu
from jax.experimental.pallas import tpu_sc as plsc

_NC = 2
_NS = 16
_NW = _NC * _NS

_CHUNK = 16
_NBUF = 2


@jax.jit
def _gather_rows(table, ids):
    B = ids.shape[0]
    V, D = table.shape
    b_per_w = B // _NW
    num_ch = b_per_w // _CHUNK
    num_groups = num_ch // _NBUF

    mesh = plsc.VectorSubcoreMesh(core_axis_name="c", subcore_axis_name="s")

    @functools.partial(
        pl.kernel,
        mesh=mesh,
        out_type=jax.ShapeDtypeStruct((B, D), jnp.float32),
        scratch_types=[
            pltpu.VMEM((b_per_w,), jnp.int32),
            pltpu.VMEM((_NBUF, _CHUNK, D), jnp.float32),
            pltpu.SemaphoreType.DMA((_NBUF,)),
            pltpu.SemaphoreType.DMA((_NBUF,)),
        ],
    )
    def k(table_hbm, idx_hbm, out_hbm, idx_v, rows_v, gsem, wsem):
        wid = lax.axis_index("s") * _NC + lax.axis_index("c")
        base = wid * b_per_w
        pltpu.sync_copy(idx_hbm.at[pl.ds(base, b_per_w)], idx_v)

        def gcopy(ch, b):
            return pltpu.make_async_copy(
                table_hbm.at[idx_v.at[pl.ds(ch * _CHUNK, _CHUNK)]],
                rows_v.at[b],
                gsem.at[b],
            )

        def wcopy(ch, b):
            return pltpu.make_async_copy(
                rows_v.at[b],
                out_hbm.at[pl.ds(base + ch * _CHUNK, _CHUNK)],
                wsem.at[b],
            )

        for b in range(_NBUF):
            gcopy(b, b).start()

        def body(j, carry):
            for b in range(_NBUF):
                ch = j * _NBUF + b
                gcopy(ch, b).wait()
                wcopy(ch, b).start()
                wcopy(ch, b).wait()
                gcopy(ch + _NBUF, b).start()
            return carry

        lax.fori_loop(0, num_groups - 1, body, 0)

        last = (num_groups - 1) * _NBUF
        for b in range(_NBUF):
            gcopy(last + b, b).wait()
            wcopy(last + b, b).start()
        for b in range(_NBUF):
            wcopy(last + b, b).wait()

    return k(table, ids)


def kernel(input_ids, embed_tokens):
    in_shape = input_ids.shape
    ids = input_ids.reshape(-1)
    out = _gather_rows(embed_tokens, ids)
    return out.reshape(in_shape + (embed_tokens.shape[1],))
